# baseline (speedup 1.0000x reference)
import jax
import jax.numpy as jnp
from jax import lax
from jax.experimental import pallas as pl
from jax.experimental.pallas import tpu as pltpu
from jax.experimental.pallas import tpu_sc as plsc

NC = 2
NS = 16
NW = NC * NS

BATCH = 4
SEQ = 2048
HIDDEN = 2048
N = BATCH * SEQ
B_PER_W = N // NW
G = 8
NG = B_PER_W // G


def _sc_lookup(indices2, table):
    mesh = plsc.VectorSubcoreMesh(core_axis_name="c", subcore_axis_name="s")

    def body(idx_hbm, table_hbm, out_hbm, idx_v, table_v, tsem, ssem0, ssem1, ssem2):
        wid = lax.axis_index("s") * NC + lax.axis_index("c")
        base = wid * B_PER_W
        pltpu.sync_copy(idx_hbm.at[wid], idx_v)
        pltpu.async_copy(table_hbm, table_v, tsem).wait()

        ssems = (ssem0, ssem1, ssem2)

        def fire(g, off, sem):
            ivec = idx_v[0, pl.ds(g * G - off, 16)]
            for k in range(G):
                pltpu.async_copy(
                    table_v.at[pl.ds(ivec[off + k], 1)],
                    out_hbm.at[pl.ds(base + g * G + k, 1)],
                    sem)

        def drain(g, sem):
            blk = out_hbm.at[pl.ds(base + g * G, G)]
            pltpu.make_async_copy(blk, blk, sem).wait()

        fire(0, 0, ssems[0])
        fire(1, 8, ssems[1])
        fire(2, 0, ssems[2])
        drain(0, ssems[0]); fire(3, 8, ssems[0])
        drain(1, ssems[1]); fire(4, 0, ssems[1])
        drain(2, ssems[2]); fire(5, 8, ssems[2])

        def loop_body(gg, carry):
            g0 = 6 * gg
            offs = (0, 8, 0, 8, 0, 8)
            for t in range(6):
                drain(g0 + t - 3, ssems[t % 3])
                fire(g0 + t, offs[t], ssems[t % 3])
            return carry

        lax.fori_loop(1, NG // 6, loop_body, 0)
        drain(NG - 5, ssems[0]); fire(NG - 2, 0, ssems[0])
        drain(NG - 4, ssems[1]); fire(NG - 1, 8, ssems[1])
        drain(NG - 3, ssems[2])
        drain(NG - 2, ssems[0])
        drain(NG - 1, ssems[1])

    run = pl.kernel(
        body,
        out_type=jax.ShapeDtypeStruct((N, HIDDEN), jnp.float32),
        mesh=mesh,
        scratch_types=[
            pltpu.VMEM((1, B_PER_W), jnp.int32),
            pltpu.VMEM((10, HIDDEN), jnp.float32),
            pltpu.SemaphoreType.DMA,
            pltpu.SemaphoreType.DMA,
            pltpu.SemaphoreType.DMA,
            pltpu.SemaphoreType.DMA,
        ],
    )
    return run(indices2, table)


def kernel(indices, table):
    idx2 = indices.astype(jnp.int32).reshape(NW, 1, B_PER_W)
    out = _sc_lookup(idx2, table)
    return out.reshape(BATCH, SEQ, HIDDEN)


# speedup vs baseline: 1.1634x; 1.1634x over previous
import jax
import jax.numpy as jnp
from jax import lax
from jax.experimental import pallas as pl
from jax.experimental.pallas import tpu as pltpu
from jax.experimental.pallas import tpu_sc as plsc

NC = 2
NS = 16
NW = NC * NS

BATCH = 4
SEQ = 2048
HIDDEN = 2048
N = BATCH * SEQ
B_PER_W = N // NW
G = 8
NG = B_PER_W // G


def _sc_lookup(indices2, table):
    mesh = plsc.VectorSubcoreMesh(core_axis_name="c", subcore_axis_name="s")

    def body(idx_hbm, table_hbm, out_hbm, idx_v, table_v, table_sh, tsem, tsem2, ssem0, ssem1):
        wid = lax.axis_index("s") * NC + lax.axis_index("c")
        base = wid * B_PER_W
        sid = lax.axis_index("s")
        pltpu.sync_copy(idx_hbm.at[wid], idx_v)

        @pl.when(sid == 0)
        def _stage_shared():
            pltpu.async_copy(table_hbm, table_sh, tsem2).wait()

        pltpu.async_copy(table_hbm, table_v, tsem).wait()
        plsc.subcore_barrier()

        ssems = (ssem0, ssem1)

        def fire(g, off, sem, src_tab):
            ivec = idx_v[0, pl.ds(g * G - off, 16)]
            for k in range(G):
                pltpu.async_copy(
                    src_tab.at[pl.ds(ivec[off + k], 1)],
                    out_hbm.at[pl.ds(base + g * G + k, 1)],
                    sem)

        def drain(g, sem):
            blk = out_hbm.at[pl.ds(base + g * G, G)]
            pltpu.make_async_copy(blk, blk, sem).wait()

        fire(0, 0, ssems[0], table_v)
        fire(1, 8, ssems[1], table_sh)

        def loop_body(gg, carry):
            g0 = 2 * gg
            drain(g0 - 2, ssems[0])
            fire(g0, 0, ssems[0], table_v)
            drain(g0 - 1, ssems[1])
            fire(g0 + 1, 8, ssems[1], table_sh)
            return carry

        lax.fori_loop(1, NG // 2, loop_body, 0)
        drain(NG - 2, ssems[0])
        drain(NG - 1, ssems[1])

    run = pl.kernel(
        body,
        out_type=jax.ShapeDtypeStruct((N, HIDDEN), jnp.float32),
        mesh=mesh,
        scratch_types=[
            pltpu.VMEM((1, B_PER_W), jnp.int32),
            pltpu.VMEM((10, HIDDEN), jnp.float32),
            pltpu.VMEM_SHARED((10, HIDDEN), jnp.float32),
            pltpu.SemaphoreType.DMA,
            pltpu.SemaphoreType.DMA,
            pltpu.SemaphoreType.DMA,
            pltpu.SemaphoreType.DMA,
        ],
    )
    return run(indices2, table)


def kernel(indices, table):
    idx2 = indices.astype(jnp.int32).reshape(NW, 1, B_PER_W)
    out = _sc_lookup(idx2, table)
    return out.reshape(BATCH, SEQ, HIDDEN)
